# Initial kernel scaffold; baseline (speedup 1.0000x reference)
import jax
import jax.numpy as jnp
from jax.experimental import pallas as pl


def kernel(flat_idx, seg, lens, W_embed, W_proj, b_proj):
    raise NotImplementedError("write your pallas kernel here")


# speedup vs baseline: 7.4866x; 7.4866x over previous
import functools

import jax
import jax.numpy as jnp
from jax import lax
from jax.experimental import pallas as pl
from jax.experimental.pallas import tpu as pltpu
from jax.experimental.pallas import tpu_sc as plsc

NUM_NEW = 100000
D_MODEL = 128
D_ENC = 256
B = 16384
N_TOK = 327680

NC = 2
NS = 16
NW = NC * NS
SEGS_PER_W = B // NW
JUNK = SEGS_PER_W
ACC_ROWS = SEGS_PER_W + 8
CHUNK = 128
PAD = 2 * CHUNK

N_ROWS = N_TOK // 128
N_RHI = N_ROWS // 128


def _offsets_body(seg_ref, out_ref):
    rl = jnp.max(seg_ref[...], axis=2)
    lane = lax.broadcasted_iota(jnp.int32, (1, 128), 1)
    acc = jnp.zeros((1, 128), jnp.int32)
    for j in range(NW + 1):
        bound = j * SEGS_PER_W
        rows_below = jnp.sum((rl < bound).astype(jnp.int32))
        ri = jnp.minimum(rows_below, N_ROWS - 1)
        row = seg_ref[ri // 128, pl.ds(ri % 128, 1), :]
        within = jnp.sum((row < bound).astype(jnp.int32))
        within = jnp.where(rows_below < N_ROWS, within, 0)
        total = rows_below * 128 + within
        acc = acc + jnp.where(lane == j, total, 0)
    out_ref[...] = acc


def _compute_offsets(seg):
    seg3d = seg.reshape(N_RHI, 128, 128)
    out = pl.pallas_call(
        _offsets_body,
        out_shape=jax.ShapeDtypeStruct((1, 128), jnp.int32),
    )(seg3d)
    return out.reshape(128)


def _read_off(offsv, i):
    return offsv[pl.ds(i, 16)][0]


def _sc_body(idx_hbm, seg_hbm, offs_hbm, table_hbm, out_hbm,
             acc_sh, idxv, segv, locv, rows, offsv, sem):
    cid = lax.axis_index("c")
    sid = lax.axis_index("s")
    w = sid * NC + cid
    seg_base = w * SEGS_PER_W
    region = sid * ACC_ROWS

    pltpu.sync_copy(offs_hbm, offsv)
    t0 = _read_off(offsv, w)
    t1 = _read_off(offsv, w + 1)
    a8 = jnp.bitwise_and(t0, jnp.int32(-8))
    nchunks = (t1 - a8 + CHUNK - 1) // CHUNK

    zero16 = jnp.zeros((16,), jnp.float32)

    def zrow(i, carry):
        for j in range(D_MODEL // 16):
            rows[i, pl.ds(j * 16, 16)] = zero16
        return carry

    lax.fori_loop(0, CHUNK, zrow, 0)
    for r in range(SEGS_PER_W // CHUNK):
        pltpu.sync_copy(rows, acc_sh.at[pl.ds(region + r * CHUNK, CHUNK)])

    def chunk_body(g, carry):
        start = pl.multiple_of(a8 + g * CHUNK, 8)
        pltpu.sync_copy(idx_hbm.at[pl.ds(start, CHUNK)], idxv)
        pltpu.sync_copy(seg_hbm.at[pl.ds(start, CHUNK)], segv)
        for j in range(CHUNK // 16):
            sv = segv[pl.ds(j * 16, 16)] - seg_base
            bad = (sv < 0) | (sv >= SEGS_PER_W)
            locv[pl.ds(j * 16, 16)] = region + jnp.where(bad, JUNK, sv)
        pltpu.async_copy(table_hbm.at[idxv], rows, sem).wait()
        pltpu.sync_copy(rows, acc_sh.at[locv], add=True)
        return carry

    lax.fori_loop(0, nchunks, chunk_body, 0)

    pltpu.sync_copy(acc_sh.at[pl.ds(region, SEGS_PER_W)],
                    out_hbm.at[pl.ds(seg_base, SEGS_PER_W)])


def _segment_sums(idx_p, seg_p, offs, W_embed):
    mesh = plsc.VectorSubcoreMesh(
        core_axis_name="c", subcore_axis_name="s",
        num_cores=NC, num_subcores=NS)
    f = pl.kernel(
        _sc_body,
        out_type=jax.ShapeDtypeStruct((B, D_MODEL), jnp.float32),
        mesh=mesh,
        scratch_types=[
            pltpu.VMEM_SHARED((NS * ACC_ROWS, D_MODEL), jnp.float32),
            pltpu.VMEM((CHUNK,), jnp.int32),
            pltpu.VMEM((CHUNK,), jnp.int32),
            pltpu.VMEM((CHUNK,), jnp.int32),
            pltpu.VMEM((CHUNK, D_MODEL), jnp.float32),
            pltpu.VMEM((128,), jnp.int32),
            pltpu.SemaphoreType.DMA,
        ],
    )
    return f(idx_p, seg_p, offs, W_embed)


def _proj_body(sums_ref, lens_ref, wp_ref, bp_ref, out_ref):
    means = sums_ref[...] * (1.0 / lens_ref[...])
    out_ref[...] = jnp.dot(means, wp_ref[...],
                           preferred_element_type=jnp.float32) + bp_ref[...]


def _project(sums, lens, W_proj, b_proj):
    blk = 512
    return pl.pallas_call(
        _proj_body,
        grid=(B // blk,),
        in_specs=[
            pl.BlockSpec((blk, D_MODEL), lambda i: (i, 0)),
            pl.BlockSpec((blk, 1), lambda i: (i, 0)),
            pl.BlockSpec((D_MODEL, D_ENC), lambda i: (0, 0)),
            pl.BlockSpec((1, D_ENC), lambda i: (0, 0)),
        ],
        out_specs=pl.BlockSpec((blk, D_ENC), lambda i: (i, 0)),
        out_shape=jax.ShapeDtypeStruct((B, D_ENC), jnp.float32),
    )(sums, lens.reshape(B, 1), W_proj, b_proj.reshape(1, D_ENC))


def kernel(flat_idx, seg, lens, W_embed, W_proj, b_proj):
    idx_p = jnp.concatenate(
        [flat_idx, jnp.zeros((PAD,), jnp.int32)])
    seg_p = jnp.concatenate(
        [seg, jnp.full((PAD,), B, jnp.int32)])
    offs = _compute_offsets(seg)
    sums = _segment_sums(idx_p, seg_p, offs, W_embed)
    return _project(sums, lens, W_proj, b_proj)


# speedup vs baseline: 13.3996x; 1.7898x over previous
import functools

import jax
import jax.numpy as jnp
from jax import lax
from jax.experimental import pallas as pl
from jax.experimental.pallas import tpu as pltpu
from jax.experimental.pallas import tpu_sc as plsc

NUM_NEW = 100000
D_MODEL = 128
D_ENC = 256
B = 16384
N_TOK = 327680

NC = 2
NS = 16
NW = NC * NS
SEGS_PER_W = B // NW
JUNK = SEGS_PER_W
ACC_ROWS = SEGS_PER_W + 8
CHUNK = 128
PAD = 4 * CHUNK

N_ROWS = N_TOK // 128
N_RHI = N_ROWS // 128


def _offsets_body(seg_ref, out_ref):
    rl = jnp.max(seg_ref[...], axis=2)
    lane = lax.broadcasted_iota(jnp.int32, (1, 128), 1)
    acc = jnp.zeros((1, 128), jnp.int32)
    for j in range(NW + 1):
        bound = j * SEGS_PER_W
        rows_below = jnp.sum((rl < bound).astype(jnp.int32))
        ri = jnp.minimum(rows_below, N_ROWS - 1)
        row = seg_ref[ri // 128, pl.ds(ri % 128, 1), :]
        within = jnp.sum((row < bound).astype(jnp.int32))
        within = jnp.where(rows_below < N_ROWS, within, 0)
        total = rows_below * 128 + within
        acc = acc + jnp.where(lane == j, total, 0)
    out_ref[...] = acc


def _compute_offsets(seg):
    seg3d = seg.reshape(N_RHI, 128, 128)
    out = pl.pallas_call(
        _offsets_body,
        out_shape=jax.ShapeDtypeStruct((1, 128), jnp.int32),
    )(seg3d)
    return out.reshape(128)


def _read_off(offsv, i):
    return offsv[pl.ds(i, 16)][0]


def _sc_body(idx_hbm, seg_hbm, offs_hbm, table_hbm, out_hbm,
             acc_sh, idxv0, idxv1, segv0, segv1, locv0, locv1,
             rows0, rows1, offsv,
             asem0, asem1, gsem0, gsem1, ssem0, ssem1):
    cid = lax.axis_index("c")
    sid = lax.axis_index("s")
    w = sid * NC + cid
    seg_base = w * SEGS_PER_W
    region = sid * ACC_ROWS

    pltpu.sync_copy(offs_hbm, offsv)
    t0 = _read_off(offsv, w)
    t1 = _read_off(offsv, w + 1)
    a8 = jnp.bitwise_and(t0, jnp.int32(-8))
    nchunks = (t1 - a8 + CHUNK - 1) // CHUNK
    n_out = jnp.maximum((nchunks + 1) // 2, 1)

    def st(g):
        return pl.multiple_of(a8 + g * CHUNK, 8)

    def issue_idxseg(start, idxv, segv, sem):
        pltpu.async_copy(idx_hbm.at[pl.ds(start, CHUNK)], idxv, sem)
        pltpu.async_copy(seg_hbm.at[pl.ds(start, CHUNK)], segv, sem)

    def wait_idxseg(idxv, segv, sem):
        pltpu.make_async_copy(idx_hbm.at[pl.ds(0, CHUNK)], idxv, sem).wait()
        pltpu.make_async_copy(seg_hbm.at[pl.ds(0, CHUNK)], segv, sem).wait()

    def compute_loc(segv, locv):
        for j in range(CHUNK // 16):
            sv = segv[pl.ds(j * 16, 16)] - seg_base
            bad = (sv < 0) | (sv >= SEGS_PER_W)
            locv[pl.ds(j * 16, 16)] = region + jnp.where(bad, JUNK, sv)

    def issue_gather(idxv, rows, sem):
        pltpu.async_copy(table_hbm.at[idxv], rows, sem)

    def wait_gather(idxv, rows, sem):
        pltpu.make_async_copy(table_hbm.at[idxv], rows, sem).wait()

    def issue_scatter(rows, locv, sem):
        pltpu.async_copy(rows, acc_sh.at[locv], sem, add=True)

    def wait_scatter(rows, locv, sem):
        pltpu.make_async_copy(rows, acc_sh.at[locv], sem).wait()

    zero16 = jnp.zeros((16,), jnp.float32)

    def zrow(i, carry):
        for j in range(D_MODEL // 16):
            rows0[i, pl.ds(j * 16, 16)] = zero16
        return carry

    lax.fori_loop(0, CHUNK, zrow, 0)
    issue_idxseg(st(0), idxv0, segv0, asem0)
    for r in range(SEGS_PER_W // CHUNK):
        pltpu.sync_copy(rows0, acc_sh.at[pl.ds(region + r * CHUNK, CHUNK)])

    wait_idxseg(idxv0, segv0, asem0)
    compute_loc(segv0, locv0)
    issue_gather(idxv0, rows0, gsem0)
    issue_idxseg(st(1), idxv1, segv1, asem1)
    wait_idxseg(idxv1, segv1, asem1)
    compute_loc(segv1, locv1)
    issue_gather(idxv1, rows1, gsem1)
    wait_gather(idxv0, rows0, gsem0)
    issue_scatter(rows0, locv0, ssem0)
    issue_idxseg(st(2), idxv0, segv0, asem0)

    def pair(i, carry):
        g0 = 2 * i
        wait_idxseg(idxv0, segv0, asem0)
        wait_scatter(rows0, locv0, ssem0)
        compute_loc(segv0, locv0)
        issue_gather(idxv0, rows0, gsem0)
        wait_gather(idxv1, rows1, gsem1)
        issue_scatter(rows1, locv1, ssem1)
        issue_idxseg(st(g0 + 1), idxv1, segv1, asem1)
        wait_idxseg(idxv1, segv1, asem1)
        wait_scatter(rows1, locv1, ssem1)
        compute_loc(segv1, locv1)
        issue_gather(idxv1, rows1, gsem1)
        wait_gather(idxv0, rows0, gsem0)
        issue_scatter(rows0, locv0, ssem0)
        issue_idxseg(st(g0 + 2), idxv0, segv0, asem0)
        return carry

    lax.fori_loop(1, n_out, pair, 0)

    wait_idxseg(idxv0, segv0, asem0)
    wait_gather(idxv1, rows1, gsem1)
    issue_scatter(rows1, locv1, ssem1)
    wait_scatter(rows0, locv0, ssem0)
    wait_scatter(rows1, locv1, ssem1)

    pltpu.sync_copy(acc_sh.at[pl.ds(region, SEGS_PER_W)],
                    out_hbm.at[pl.ds(seg_base, SEGS_PER_W)])


def _segment_sums(idx_p, seg_p, offs, W_embed):
    mesh = plsc.VectorSubcoreMesh(
        core_axis_name="c", subcore_axis_name="s",
        num_cores=NC, num_subcores=NS)
    f = pl.kernel(
        _sc_body,
        out_type=jax.ShapeDtypeStruct((B, D_MODEL), jnp.float32),
        mesh=mesh,
        scratch_types=[
            pltpu.VMEM_SHARED((NS * ACC_ROWS, D_MODEL), jnp.float32),
            pltpu.VMEM((CHUNK,), jnp.int32),
            pltpu.VMEM((CHUNK,), jnp.int32),
            pltpu.VMEM((CHUNK,), jnp.int32),
            pltpu.VMEM((CHUNK,), jnp.int32),
            pltpu.VMEM((CHUNK,), jnp.int32),
            pltpu.VMEM((CHUNK,), jnp.int32),
            pltpu.VMEM((CHUNK, D_MODEL), jnp.float32),
            pltpu.VMEM((CHUNK, D_MODEL), jnp.float32),
            pltpu.VMEM((128,), jnp.int32),
            pltpu.SemaphoreType.DMA,
            pltpu.SemaphoreType.DMA,
            pltpu.SemaphoreType.DMA,
            pltpu.SemaphoreType.DMA,
            pltpu.SemaphoreType.DMA,
            pltpu.SemaphoreType.DMA,
        ],
    )
    return f(idx_p, seg_p, offs, W_embed)


def _proj_body(sums_ref, lens_ref, wp_ref, bp_ref, out_ref):
    means = sums_ref[...] * (1.0 / lens_ref[...])
    out_ref[...] = jnp.dot(means, wp_ref[...],
                           preferred_element_type=jnp.float32) + bp_ref[...]


def _project(sums, lens, W_proj, b_proj):
    blk = 512
    return pl.pallas_call(
        _proj_body,
        grid=(B // blk,),
        in_specs=[
            pl.BlockSpec((blk, D_MODEL), lambda i: (i, 0)),
            pl.BlockSpec((blk, 1), lambda i: (i, 0)),
            pl.BlockSpec((D_MODEL, D_ENC), lambda i: (0, 0)),
            pl.BlockSpec((1, D_ENC), lambda i: (0, 0)),
        ],
        out_specs=pl.BlockSpec((blk, D_ENC), lambda i: (i, 0)),
        out_shape=jax.ShapeDtypeStruct((B, D_ENC), jnp.float32),
    )(sums, lens.reshape(B, 1), W_proj, b_proj.reshape(1, D_ENC))


def kernel(flat_idx, seg, lens, W_embed, W_proj, b_proj):
    idx_p = jnp.concatenate(
        [flat_idx, jnp.zeros((PAD,), jnp.int32)])
    seg_p = jnp.concatenate(
        [seg, jnp.full((PAD,), B, jnp.int32)])
    offs = _compute_offsets(seg)
    sums = _segment_sums(idx_p, seg_p, offs, W_embed)
    return _project(sums, lens, W_proj, b_proj)
